# baseline (speedup 1.0000x reference)
import functools

import jax
import jax.numpy as jnp
from jax import lax
from jax.experimental import pallas as pl
from jax.experimental.pallas import tpu as pltpu
from jax.experimental.pallas import tpu_sc as plsc

N_NODES = 10000
N_EDGES = 320000
D = 128
EPS = 1e-5

NC = 2
NS = 16
CHUNK = 128
CHUNKS_PER_TILE = 80
IDX_BLK = 16
BLK_C0 = 2
BLK_C1 = 8
C0_ROWS = NS * BLK_C0 * IDX_BLK
E_PAD = NC * NS * CHUNKS_PER_TILE * CHUNK
ROWS_PAD = 10112
CW = 16
R_PER_TILE = ROWS_PAD // NS


def _sc_aggregate(x, src2, dst2, zfeat, zcnt):
    mesh = plsc.VectorSubcoreMesh(core_axis_name="c", subcore_axis_name="s")

    @functools.partial(
        pl.kernel,
        mesh=mesh,
        compiler_params=pltpu.CompilerParams(needs_layout_passes=False),
        out_type=[
            jax.ShapeDtypeStruct((NC, ROWS_PAD, D), jnp.float32),
            jax.ShapeDtypeStruct((NC, NS, ROWS_PAD), jnp.float32),
        ],
        scratch_types=[
            pltpu.VMEM_SHARED((ROWS_PAD, D), jnp.float32),
            pltpu.VMEM((IDX_BLK, CHUNK), jnp.int32),
            pltpu.VMEM((IDX_BLK, 1, CHUNK), jnp.int32),
            pltpu.VMEM((CHUNK, D), jnp.float32),
            pltpu.VMEM((CHUNK, D), jnp.float32),
            pltpu.VMEM((ROWS_PAD,), jnp.float32),
            pltpu.SemaphoreType.DMA,
            pltpu.SemaphoreType.DMA,
        ],
    )
    def k(x_hbm, src_hbm, dst_hbm, zf_hbm, zc_hbm, agg_out, cnt_out,
          agg_sh, src_v, dst_v, rows_a, rows_b, cnt_v, sem_a, sem_b):
        cid = lax.axis_index("c")
        sid = lax.axis_index("s")
        wid = cid * NS + sid

        r0 = sid * R_PER_TILE
        pltpu.sync_copy(zf_hbm, agg_sh.at[pl.ds(r0, R_PER_TILE)])
        pltpu.sync_copy(zc_hbm, cnt_v)

        base_rows = jnp.where(cid == 0, sid * (BLK_C0 * IDX_BLK),
                              C0_ROWS + sid * (BLK_C1 * IDX_BLK))
        n_blk = jnp.where(cid == 0, BLK_C0, BLK_C1)

        plsc.subcore_barrier()

        def _gather(j, buf, sem):
            pltpu.async_copy(x_hbm.at[src_v.at[j]], buf, sem)

        def _drain(buf, sem):
            pltpu.make_async_copy(x_hbm.at[pl.ds(0, CHUNK)], buf, sem).wait()

        def _hist(j):
            ones16 = jnp.full((16,), 1.0, jnp.float32)
            for kk in range(CHUNK // 16):
                idx16 = dst_v[j, 0, pl.ds(kk * 16, 16)]
                plsc.addupdate_scatter(cnt_v, [idx16], ones16)

        def _scatter(j, buf):
            pltpu.sync_copy(buf, agg_sh.at[dst_v.at[j, 0]], add=True)

        def _idx_block(b, _):
            pltpu.sync_copy(src_hbm.at[pl.ds(base_rows + b * IDX_BLK, IDX_BLK)],
                            src_v)
            pltpu.sync_copy(dst_hbm.at[pl.ds(base_rows + b * IDX_BLK, IDX_BLK)],
                            dst_v)
            _gather(0, rows_a, sem_a)

            def _pair(k2, _):
                _gather(2 * k2 + 1, rows_b, sem_b)
                _hist(2 * k2)
                _drain(rows_a, sem_a)
                _scatter(2 * k2, rows_a)

                @pl.when(k2 < IDX_BLK // 2 - 1)
                def _next():
                    _gather(2 * k2 + 2, rows_a, sem_a)
                _hist(2 * k2 + 1)
                _drain(rows_b, sem_b)
                _scatter(2 * k2 + 1, rows_b)
                return 0
            return lax.fori_loop(0, IDX_BLK // 2, _pair, 0)
        lax.fori_loop(0, n_blk, _idx_block, 0)

        plsc.subcore_barrier()

        pltpu.sync_copy(agg_sh.at[pl.ds(r0, R_PER_TILE)],
                        agg_out.at[cid, pl.ds(r0, R_PER_TILE)])
        pltpu.sync_copy(cnt_v, cnt_out.at[cid, sid])

    return k(x, src2, dst2, zfeat, zcnt)


RB = 1000
GRID = N_NODES // RB


def _tc_h_body(x_ref, agg_ref, cnt_ref, wl_ref, bl_ref, wr_ref, h_ref, s_ref):
    i = pl.program_id(0)
    agg = agg_ref[0] + agg_ref[1]
    cnt = jnp.sum(cnt_ref[...], axis=1, keepdims=True)
    mean = agg / jnp.maximum(cnt, 1.0)
    dn = (((1,), (1,)), ((), ()))
    h = (lax.dot_general(mean, wl_ref[...], dn,
                         preferred_element_type=jnp.float32,
                         precision=lax.Precision.HIGHEST)
         + bl_ref[...][None, :]
         + lax.dot_general(x_ref[...], wr_ref[...], dn,
                           preferred_element_type=jnp.float32,
                           precision=lax.Precision.HIGHEST))
    h_ref[...] = h
    part = jnp.concatenate(
        [jnp.sum(h, axis=0, keepdims=True),
         jnp.sum(h * h, axis=0, keepdims=True)], axis=0)

    @pl.when(i == 0)
    def _init():
        s_ref[...] = part

    @pl.when(i != 0)
    def _acc():
        s_ref[...] = s_ref[...] + part


def _tc_norm_body(h_ref, s_ref, g_ref, b_ref, o_ref):
    mu = s_ref[0:1, :] * (1.0 / N_NODES)
    var = s_ref[1:2, :] * (1.0 / N_NODES) - mu * mu
    y = (g_ref[...][None, :] * (h_ref[...] - mu) * lax.rsqrt(var + EPS)
         + b_ref[...][None, :])
    o_ref[...] = jnp.maximum(y, 0.0)


def _tc_combine(x, aggp, cntp, W_l, b_l, W_r, gamma, beta):
    h, sums = pl.pallas_call(
        _tc_h_body,
        grid=(GRID,),
        in_specs=[
            pl.BlockSpec((RB, D), lambda i: (i, 0)),
            pl.BlockSpec((NC, RB, D), lambda i: (0, i, 0)),
            pl.BlockSpec((RB, NC * NS), lambda i: (i, 0)),
            pl.BlockSpec((D, D), lambda i: (0, 0)),
            pl.BlockSpec((D,), lambda i: (0,)),
            pl.BlockSpec((D, D), lambda i: (0, 0)),
        ],
        out_specs=[
            pl.BlockSpec((RB, D), lambda i: (i, 0)),
            pl.BlockSpec((2, D), lambda i: (0, 0)),
        ],
        out_shape=[
            jax.ShapeDtypeStruct((N_NODES, D), jnp.float32),
            jax.ShapeDtypeStruct((2, D), jnp.float32),
        ],
    )(x, aggp, cntp, W_l, b_l, W_r)
    return pl.pallas_call(
        _tc_norm_body,
        grid=(GRID,),
        in_specs=[
            pl.BlockSpec((RB, D), lambda i: (i, 0)),
            pl.BlockSpec((2, D), lambda i: (0, 0)),
            pl.BlockSpec((D,), lambda i: (0,)),
            pl.BlockSpec((D,), lambda i: (0,)),
        ],
        out_specs=pl.BlockSpec((RB, D), lambda i: (i, 0)),
        out_shape=jax.ShapeDtypeStruct((N_NODES, D), jnp.float32),
    )(h, sums, gamma, beta)


def kernel(x, edge_index, W_l, b_l, W_r, gamma, beta):
    src = edge_index[0].astype(jnp.int32)
    dst = edge_index[1].astype(jnp.int32)
    pad = E_PAD - N_EDGES
    src2 = jnp.concatenate([src, jnp.zeros((pad,), jnp.int32)])
    src2 = src2.reshape(E_PAD // CHUNK, CHUNK)
    dst2 = jnp.concatenate([dst, jnp.full((pad,), N_NODES, jnp.int32)])
    dst2 = dst2.reshape(E_PAD // CHUNK, 1, CHUNK)
    zfeat = jnp.zeros((R_PER_TILE, D), jnp.float32)
    zcnt = jnp.zeros((ROWS_PAD,), jnp.float32)
    aggp, cntp = _sc_aggregate(x, src2, dst2, zfeat, zcnt)
    cnt_t = jnp.transpose(cntp.reshape(NC * NS, ROWS_PAD))
    return _tc_combine(x, aggp, cnt_t, W_l, b_l, W_r, gamma, beta)


# speedup vs baseline: 1.0856x; 1.0856x over previous
import functools

import jax
import jax.numpy as jnp
from jax import lax
from jax.experimental import pallas as pl
from jax.experimental.pallas import tpu as pltpu
from jax.experimental.pallas import tpu_sc as plsc

N_NODES = 10000
N_EDGES = 320000
D = 128
EPS = 1e-5

NC = 2
NS = 16
CHUNK = 128
CHUNKS_PER_TILE = 80
IDX_BLK = 16
N_BLK = CHUNKS_PER_TILE // IDX_BLK
E_PAD = NC * NS * CHUNKS_PER_TILE * CHUNK
ROWS_PAD = 10112
R_PER_TILE = ROWS_PAD // NS


def _sc_aggregate(x, src2, dst2, zfeat, zcnt):
    mesh = plsc.VectorSubcoreMesh(core_axis_name="c", subcore_axis_name="s")

    @functools.partial(
        pl.kernel,
        mesh=mesh,
        compiler_params=pltpu.CompilerParams(needs_layout_passes=False),
        out_type=[
            jax.ShapeDtypeStruct((NC, ROWS_PAD, D), jnp.float32),
            jax.ShapeDtypeStruct((NC, NS, ROWS_PAD), jnp.float32),
        ],
        scratch_types=[
            pltpu.VMEM_SHARED((ROWS_PAD, D), jnp.float32),
            pltpu.VMEM((IDX_BLK, CHUNK), jnp.int32),
            pltpu.VMEM((IDX_BLK, 1, CHUNK), jnp.int32),
            pltpu.VMEM((CHUNK, D), jnp.float32),
            pltpu.VMEM((CHUNK, D), jnp.float32),
            pltpu.VMEM((ROWS_PAD,), jnp.float32),
            pltpu.SemaphoreType.DMA,
            pltpu.SemaphoreType.DMA,
        ],
    )
    def k(x_hbm, src_hbm, dst_hbm, zf_hbm, zc_hbm, agg_out, cnt_out,
          agg_sh, src_v, dst_v, rows_a, rows_b, cnt_v, sem_a, sem_b):
        cid = lax.axis_index("c")
        sid = lax.axis_index("s")
        wid = cid * NS + sid

        r0 = sid * R_PER_TILE
        pltpu.sync_copy(zf_hbm, agg_sh.at[pl.ds(r0, R_PER_TILE)])
        pltpu.sync_copy(zc_hbm, cnt_v)

        base_rows = wid * CHUNKS_PER_TILE

        plsc.subcore_barrier()

        def _gather(j, buf, sem):
            pltpu.async_copy(x_hbm.at[src_v.at[j]], buf, sem)

        def _drain(buf, sem):
            pltpu.make_async_copy(x_hbm.at[pl.ds(0, CHUNK)], buf, sem).wait()

        def _hist(j):
            ones16 = jnp.full((16,), 1.0, jnp.float32)
            for kk in range(CHUNK // 16):
                idx16 = dst_v[j, 0, pl.ds(kk * 16, 16)]
                plsc.addupdate_scatter(cnt_v, [idx16], ones16)

        def _scatter(j, buf):
            pltpu.sync_copy(buf, agg_sh.at[dst_v.at[j, 0]], add=True)

        def _idx_block(b, _):
            pltpu.sync_copy(src_hbm.at[pl.ds(base_rows + b * IDX_BLK, IDX_BLK)],
                            src_v)
            pltpu.sync_copy(dst_hbm.at[pl.ds(base_rows + b * IDX_BLK, IDX_BLK)],
                            dst_v)
            _gather(0, rows_a, sem_a)

            def _pair(k2, _):
                _gather(2 * k2 + 1, rows_b, sem_b)
                _hist(2 * k2)
                _drain(rows_a, sem_a)
                _scatter(2 * k2, rows_a)

                @pl.when(k2 < IDX_BLK // 2 - 1)
                def _next():
                    _gather(2 * k2 + 2, rows_a, sem_a)
                _hist(2 * k2 + 1)
                _drain(rows_b, sem_b)
                _scatter(2 * k2 + 1, rows_b)
                return 0
            return lax.fori_loop(0, IDX_BLK // 2, _pair, 0)
        lax.fori_loop(0, N_BLK, _idx_block, 0)

        plsc.subcore_barrier()

        pltpu.sync_copy(agg_sh.at[pl.ds(r0, R_PER_TILE)],
                        agg_out.at[cid, pl.ds(r0, R_PER_TILE)])
        pltpu.sync_copy(cnt_v, cnt_out.at[cid, sid])

    return k(x, src2, dst2, zfeat, zcnt)


RB = 1000
GRID = N_NODES // RB


def _tc_h_body(x_ref, agg_ref, cnt_ref, wl_ref, bl_ref, wr_ref, h_ref, s_ref):
    i = pl.program_id(0)
    agg = agg_ref[0] + agg_ref[1]
    cnt = jnp.sum(cnt_ref[...], axis=1, keepdims=True)
    mean = agg / jnp.maximum(cnt, 1.0)
    dn = (((1,), (1,)), ((), ()))
    h = (lax.dot_general(mean, wl_ref[...], dn,
                         preferred_element_type=jnp.float32,
                         precision=lax.Precision.HIGHEST)
         + bl_ref[...][None, :]
         + lax.dot_general(x_ref[...], wr_ref[...], dn,
                           preferred_element_type=jnp.float32,
                           precision=lax.Precision.HIGHEST))
    h_ref[...] = h
    part = jnp.concatenate(
        [jnp.sum(h, axis=0, keepdims=True),
         jnp.sum(h * h, axis=0, keepdims=True)], axis=0)

    @pl.when(i == 0)
    def _init():
        s_ref[...] = part

    @pl.when(i != 0)
    def _acc():
        s_ref[...] = s_ref[...] + part


def _tc_norm_body(h_ref, s_ref, g_ref, b_ref, o_ref):
    mu = s_ref[0:1, :] * (1.0 / N_NODES)
    var = s_ref[1:2, :] * (1.0 / N_NODES) - mu * mu
    y = (g_ref[...][None, :] * (h_ref[...] - mu) * lax.rsqrt(var + EPS)
         + b_ref[...][None, :])
    o_ref[...] = jnp.maximum(y, 0.0)


def _tc_combine(x, aggp, cntp, W_l, b_l, W_r, gamma, beta):
    h, sums = pl.pallas_call(
        _tc_h_body,
        grid=(GRID,),
        in_specs=[
            pl.BlockSpec((RB, D), lambda i: (i, 0)),
            pl.BlockSpec((NC, RB, D), lambda i: (0, i, 0)),
            pl.BlockSpec((RB, NC * NS), lambda i: (i, 0)),
            pl.BlockSpec((D, D), lambda i: (0, 0)),
            pl.BlockSpec((D,), lambda i: (0,)),
            pl.BlockSpec((D, D), lambda i: (0, 0)),
        ],
        out_specs=[
            pl.BlockSpec((RB, D), lambda i: (i, 0)),
            pl.BlockSpec((2, D), lambda i: (0, 0)),
        ],
        out_shape=[
            jax.ShapeDtypeStruct((N_NODES, D), jnp.float32),
            jax.ShapeDtypeStruct((2, D), jnp.float32),
        ],
    )(x, aggp, cntp, W_l, b_l, W_r)
    return pl.pallas_call(
        _tc_norm_body,
        grid=(GRID,),
        in_specs=[
            pl.BlockSpec((RB, D), lambda i: (i, 0)),
            pl.BlockSpec((2, D), lambda i: (0, 0)),
            pl.BlockSpec((D,), lambda i: (0,)),
            pl.BlockSpec((D,), lambda i: (0,)),
        ],
        out_specs=pl.BlockSpec((RB, D), lambda i: (i, 0)),
        out_shape=jax.ShapeDtypeStruct((N_NODES, D), jnp.float32),
    )(h, sums, gamma, beta)


def kernel(x, edge_index, W_l, b_l, W_r, gamma, beta):
    src = edge_index[0].astype(jnp.int32)
    dst = edge_index[1].astype(jnp.int32)
    pad = E_PAD - N_EDGES
    src2 = jnp.concatenate([src, jnp.zeros((pad,), jnp.int32)])
    src2 = src2.reshape(E_PAD // CHUNK, CHUNK)
    dst2 = jnp.concatenate([dst, jnp.full((pad,), N_NODES, jnp.int32)])
    dst2 = dst2.reshape(E_PAD // CHUNK, 1, CHUNK)
    zfeat = jnp.zeros((R_PER_TILE, D), jnp.float32)
    zcnt = jnp.zeros((ROWS_PAD,), jnp.float32)
    aggp, cntp = _sc_aggregate(x, src2, dst2, zfeat, zcnt)
    cnt_t = jnp.transpose(cntp.reshape(NC * NS, ROWS_PAD))
    return _tc_combine(x, aggp, cnt_t, W_l, b_l, W_r, gamma, beta)
